# baseline (speedup 1.0000x reference)
import jax
import jax.numpy as jnp
from jax import lax
from jax.experimental import pallas as pl
from jax.experimental.pallas import tpu as pltpu
from jax.experimental.pallas import tpu_sc as plsc

_N = 10000
_E = 320000
_D = 128
_H = 64
_Z = 32

_NC = 2
_NS = 16
_NW = _NC * _NS
_EPW = _E // _NW
_C = 125
_NCH = _EPW // _C
_NSLOT = 4
_RPT = 624
_TAIL = _N - _RPT * _NS
_W = 32

_DW = 8
_GPT = _RPT // 16
_RROW = 40
_RD = _NS * _RROW + 8
_ZR = 104


def _sc_agg(ys, src3, dst3, rdeg_in):
    mesh = plsc.VectorSubcoreMesh(core_axis_name="c", subcore_axis_name="s")
    with_deg = rdeg_in is None
    nph = len(ys)

    out_type = [jax.ShapeDtypeStruct((nph, _NC, _N, _W), jnp.float32)]
    if with_deg:
        out_type.append(jax.ShapeDtypeStruct((_NC, _RD, 16), jnp.float32))

    scratch = [
        pltpu.VMEM((_NCH, _C), jnp.int32),
        pltpu.VMEM((2 if with_deg else 1, _NCH, _C), jnp.int32),
        [pltpu.VMEM((_C, _W), jnp.float32)] * _NSLOT,
        pltpu.VMEM((_ZR, _W), jnp.float32),
        pltpu.VMEM((_C, _DW), jnp.float32),
        pltpu.VMEM((_RPT, _W), jnp.float32),
        pltpu.VMEM((_RPT, _DW), jnp.float32),
        pltpu.VMEM((_RROW, 16), jnp.float32),
        pltpu.VMEM_SHARED((_N, _W), jnp.float32),
        [pltpu.SemaphoreType.DMA] * _NSLOT,
        [pltpu.SemaphoreType.DMA] * _NSLOT,
    ]
    if with_deg:
        scratch.append(
            pltpu.VMEM_SHARED((_N, _DW), jnp.float32))

    def body(*args):
        y_hbms = args[:nph]
        (src_hbm, dst_hbm, zdeg_hbm, ones_hbm, rdeg_hbm) = args[nph:nph + 5]
        refs = args[nph + 5:]
        if with_deg:
            (out_hbm, rdeg_out, src_v, dst_v, bufs, zbuf, ones_v, av, dv,
             rv, acc, gsem, ssem, dacc) = refs
        else:
            (out_hbm, src_v, dst_v, bufs, zbuf, ones_v, av, dv,
             rv, acc, gsem, ssem) = refs
            dacc = rdeg_out = None
        cid = lax.axis_index("c")
        sid = lax.axis_index("s")
        r0 = sid * _RPT
        t0 = _RPT * _NS
        data_slab = cid if with_deg else 0
        iota16 = lax.iota(jnp.int32, 16)
        zeros16 = jnp.zeros((16,), jnp.int32)

        @pl.loop(0, _ZR)
        def _fill(i):
            for c in range(_W // 16):
                zbuf[i, pl.ds(c * 16, 16)] = jnp.zeros((16,), jnp.float32)

        pltpu.sync_copy(src_hbm.at[2 * sid + cid], src_v)
        if with_deg:
            pltpu.sync_copy(dst_hbm.at[pl.ds(2 * sid, 2)], dst_v)
            pltpu.sync_copy(ones_hbm, ones_v)
            pltpu.sync_copy(zdeg_hbm.at[pl.ds(r0, _RPT)],
                            dacc.at[pl.ds(r0, _RPT)])

            @pl.when(sid == 0)
            def _zdtail():
                pltpu.sync_copy(zdeg_hbm.at[pl.ds(t0, _TAIL)],
                                dacc.at[pl.ds(t0, _TAIL)])
        else:
            pltpu.sync_copy(dst_hbm.at[pl.ds(2 * sid + cid, 1)], dst_v)
            pltpu.sync_copy(rdeg_hbm.at[cid].at[pl.ds(_RROW * sid, _GPT)],
                            rv.at[pl.ds(0, _GPT)])

        def issue_gather(y_hbm, k, j):
            pltpu.async_copy(y_hbm.at[src_v.at[k]], bufs[j], gsem[j])

        def issue_scatter(k, j, deg_too):
            pltpu.async_copy(bufs[j], acc.at[dst_v.at[data_slab].at[k]],
                             ssem[j], add=True)
            if deg_too:
                for w in range(2):
                    pltpu.async_copy(ones_v, dacc.at[dst_v.at[w].at[k]],
                                     ssem[j], add=True)

        def wait_gather(y_hbm, j):
            pltpu.make_async_copy(y_hbm.at[pl.ds(0, _C)], bufs[j],
                                  gsem[j]).wait()

        def wait_scatter(y_hbm, j, deg_too):
            pltpu.make_async_copy(y_hbm.at[pl.ds(0, _C)], bufs[j],
                                  ssem[j]).wait()
            if deg_too:
                for _ in range(2):
                    pltpu.make_async_copy(ones_hbm, ones_v, ssem[j]).wait()

        def scale_rows(row0, rvoff, ngroups, compute_recip):
            pltpu.sync_copy(acc.at[pl.ds(row0, ngroups * 16)],
                            av.at[pl.ds(0, ngroups * 16)])
            if compute_recip:
                pltpu.sync_copy(dacc.at[pl.ds(row0, ngroups * 16)],
                                dv.at[pl.ds(0, ngroups * 16)])

            @pl.loop(0, ngroups)
            def grp(g):
                if compute_recip:
                    dvals = plsc.load_gather(dv, [g * 16 + iota16, zeros16])
                    rv[rvoff + g] = 1.0 / jnp.maximum(dvals, 1.0)

                @pl.loop(0, 16)
                def node(t):
                    s = plsc.load_gather(
                        rv, [jnp.full((16,), rvoff + g, jnp.int32),
                             jnp.full((16,), t, jnp.int32)])
                    row = g * 16 + t
                    for c in range(_W // 16):
                        sl = pl.ds(c * 16, 16)
                        av[row, sl] = av[row, sl] * s

        for p in range(nph):
            y_hbm = y_hbms[p]
            deg_too = with_deg and p == 0
            first_recip = with_deg and p == 0

            for t in range(_RPT // _ZR):
                pltpu.sync_copy(zbuf, acc.at[pl.ds(r0 + t * _ZR, _ZR)])

            @pl.when(sid == 0)
            def _zero_tail():
                pltpu.sync_copy(zbuf.at[pl.ds(0, _TAIL)],
                                acc.at[pl.ds(t0, _TAIL)])

            plsc.subcore_barrier()

            for j in range(_NSLOT):
                issue_gather(y_hbm, j, j)

            @pl.loop(0, (_NCH - _NSLOT) // _NSLOT)
            def group(h):
                base = h * _NSLOT
                for bank in (0, 1):
                    for t in (0, 1):
                        j = 2 * bank + t
                        wait_gather(y_hbm, j)
                        issue_scatter(base + j, j, deg_too)
                    for t in (0, 1):
                        j = 2 * bank + t
                        wait_scatter(y_hbm, j, deg_too)
                        issue_gather(y_hbm, base + _NSLOT + j, j)

            for j in range(_NSLOT):
                wait_gather(y_hbm, j)
                issue_scatter(_NCH - _NSLOT + j, j, deg_too)
            for j in range(_NSLOT):
                wait_scatter(y_hbm, j, deg_too)

            plsc.subcore_barrier()

            scale_rows(r0, 0, _GPT, first_recip)
            pltpu.sync_copy(av, out_hbm.at[p].at[cid].at[pl.ds(r0, _RPT)])
            if first_recip:
                pltpu.sync_copy(
                    rv.at[pl.ds(0, _GPT)],
                    rdeg_out.at[cid].at[pl.ds(_RROW * sid, _GPT)])

            @pl.when(sid == 0)
            def _tail():
                if first_recip:
                    scale_rows(t0, _GPT, _TAIL // 16, True)
                    pltpu.sync_copy(
                        rv.at[pl.ds(_GPT, _TAIL // 16)],
                        rdeg_out.at[cid].at[pl.ds(_NS * _RROW,
                                                  _TAIL // 16)])
                else:
                    if not with_deg and p == 0:
                        pltpu.sync_copy(
                            rdeg_hbm.at[cid].at[pl.ds(_NS * _RROW, 1)],
                            rv.at[pl.ds(_GPT, 1)])
                    scale_rows(t0, _GPT, _TAIL // 16, False)
                pltpu.sync_copy(av.at[pl.ds(0, _TAIL)],
                                out_hbm.at[p].at[cid].at[pl.ds(t0, _TAIL)])

    zdeg = jnp.zeros((_N, _DW), jnp.float32)
    ones = jnp.ones((_C, _DW), jnp.float32)
    if rdeg_in is None:
        rdeg_in = jnp.zeros((_NC, _RD, 16), jnp.float32)
    k = pl.kernel(body, out_type=out_type, mesh=mesh, scratch_types=scratch,
                  compiler_params=pltpu.CompilerParams(
                      use_tc_tiling_on_sc=False,
                      needs_layout_passes=False))
    return k(*ys, src3, dst3, zdeg, ones, rdeg_in)


def _tc1_body(x_ref, wl_ref, wr_ref, y1a_ref, y1b_ref, r1_ref):
    xb = x_ref[...]
    y1 = jnp.dot(xb, wl_ref[...], preferred_element_type=jnp.float32)
    y1a_ref[...] = y1[:, :_W]
    y1b_ref[...] = y1[:, _W:]
    r1_ref[...] = jnp.dot(xb, wr_ref[...], preferred_element_type=jnp.float32)


def _tc2_body(pa0_ref, pa1_ref, pb0_ref, pb1_ref, r1_ref, b1_ref,
              wl2_ref, wr2_ref, y2_ref, r2_ref):
    mean_a = (pa0_ref[...] + pa1_ref[...]).reshape(_N, _W)
    mean_b = (pb0_ref[...] + pb1_ref[...]).reshape(_N, _W)
    mean = jnp.concatenate([mean_a, mean_b], axis=1)
    h = jnp.maximum(mean + r1_ref[...] + b1_ref[...], 0.0)
    y2_ref[...] = jnp.dot(h, wl2_ref[...], preferred_element_type=jnp.float32)
    r2_ref[...] = jnp.dot(h, wr2_ref[...], preferred_element_type=jnp.float32)


def _tc3_body(pa_ref, pb_ref, r2_ref, b2_ref, z_ref):
    mean = (pa_ref[...] + pb_ref[...]).reshape(_N, _Z)
    z_ref[...] = jnp.maximum(mean + r2_ref[...] + b2_ref[...], 0.0)


def kernel(x, edge_index, W_l1, W_r1, b1, W_l2, W_r2, b2):
    src3 = edge_index[0].reshape(_NW, _NCH, _C)
    dst3 = edge_index[1].reshape(_NW, _NCH, _C)

    y1a, y1b, r1 = pl.pallas_call(
        _tc1_body,
        grid=(1,),
        in_specs=[
            pl.BlockSpec((_N, _D), lambda i: (0, 0)),
            pl.BlockSpec((_D, _H), lambda i: (0, 0)),
            pl.BlockSpec((_D, _H), lambda i: (0, 0)),
        ],
        out_specs=[
            pl.BlockSpec((_N, _W), lambda i: (0, 0)),
            pl.BlockSpec((_N, _W), lambda i: (0, 0)),
            pl.BlockSpec((_N, _H), lambda i: (0, 0)),
        ],
        out_shape=[
            jax.ShapeDtypeStruct((_N, _W), jnp.float32),
            jax.ShapeDtypeStruct((_N, _W), jnp.float32),
            jax.ShapeDtypeStruct((_N, _H), jnp.float32),
        ],
    )(x, W_l1, W_r1)

    pm1, rdeg = _sc_agg([y1a, y1b], src3, dst3, None)

    y2, r2 = pl.pallas_call(
        _tc2_body,
        grid=(1,),
        in_specs=[
            pl.BlockSpec((1, 1, _N, _W), lambda i: (0, 0, 0, 0)),
            pl.BlockSpec((1, 1, _N, _W), lambda i: (0, 1, 0, 0)),
            pl.BlockSpec((1, 1, _N, _W), lambda i: (1, 0, 0, 0)),
            pl.BlockSpec((1, 1, _N, _W), lambda i: (1, 1, 0, 0)),
            pl.BlockSpec((_N, _H), lambda i: (0, 0)),
            pl.BlockSpec((1, _H), lambda i: (0, 0)),
            pl.BlockSpec((_H, _Z), lambda i: (0, 0)),
            pl.BlockSpec((_H, _Z), lambda i: (0, 0)),
        ],
        out_specs=[
            pl.BlockSpec((_N, _Z), lambda i: (0, 0)),
            pl.BlockSpec((_N, _Z), lambda i: (0, 0)),
        ],
        out_shape=[
            jax.ShapeDtypeStruct((_N, _Z), jnp.float32),
            jax.ShapeDtypeStruct((_N, _Z), jnp.float32),
        ],
    )(pm1, pm1, pm1, pm1, r1, b1.reshape(1, _H), W_l2, W_r2)

    pm2, = _sc_agg([y2], src3, dst3, rdeg)

    z = pl.pallas_call(
        _tc3_body,
        grid=(1,),
        in_specs=[
            pl.BlockSpec((1, 1, _N, _Z), lambda i: (0, 0, 0, 0)),
            pl.BlockSpec((1, 1, _N, _Z), lambda i: (0, 1, 0, 0)),
            pl.BlockSpec((_N, _Z), lambda i: (0, 0)),
            pl.BlockSpec((1, _Z), lambda i: (0, 0)),
        ],
        out_specs=pl.BlockSpec((_N, _Z), lambda i: (0, 0)),
        out_shape=jax.ShapeDtypeStruct((_N, _Z), jnp.float32),
    )(pm2, pm2, r2, b2.reshape(1, _Z))

    return z


# speedup vs baseline: 1.1673x; 1.1673x over previous
import jax
import jax.numpy as jnp
from jax import lax
from jax.experimental import pallas as pl
from jax.experimental.pallas import tpu as pltpu
from jax.experimental.pallas import tpu_sc as plsc

_N = 10000
_E = 320000
_D = 128
_H = 64
_Z = 32

_NC = 2
_NS = 16
_NW = _NC * _NS
_EPW = _E // _NW
_C = 125
_NCH = _EPW // _C
_NSLOT = 4
_RPT = 624
_TAIL = _N - _RPT * _NS
_W = 32

_DW = 8
_GPT = _RPT // 16
_RROW = 40
_RD = _NS * _RROW + 8
_ZR = 104


def _sc_agg(ys, src3, dst3, rdeg_in):
    mesh = plsc.VectorSubcoreMesh(core_axis_name="c", subcore_axis_name="s")
    with_deg = rdeg_in is None
    nph = len(ys)

    out_type = [jax.ShapeDtypeStruct((nph, _NC, _N, _W), jnp.float32)]
    if with_deg:
        out_type.append(jax.ShapeDtypeStruct((_NC, _RD, 16), jnp.float32))

    scratch = [
        pltpu.VMEM((_NCH, _C), jnp.int32),
        pltpu.VMEM((2 if with_deg else 1, _NCH, _C), jnp.int32),
        [pltpu.VMEM((_C, _W), jnp.float32)] * _NSLOT,
        pltpu.VMEM((_ZR, _W), jnp.float32),
        pltpu.VMEM((_C, _DW), jnp.float32),
        pltpu.VMEM((_RPT, _W), jnp.float32),
        pltpu.VMEM((_RPT, _DW), jnp.float32),
        pltpu.VMEM((_RROW, 16), jnp.float32),
        pltpu.VMEM_SHARED((_N, _W), jnp.float32),
        [pltpu.SemaphoreType.DMA] * _NSLOT,
        [pltpu.SemaphoreType.DMA] * _NSLOT,
    ]
    if with_deg:
        scratch.append(
            pltpu.VMEM_SHARED((_N, _DW), jnp.float32))

    def body(*args):
        y_hbms = args[:nph]
        (src_hbm, dst_hbm, zdeg_hbm, ones_hbm, rdeg_hbm) = args[nph:nph + 5]
        refs = args[nph + 5:]
        if with_deg:
            (out_hbm, rdeg_out, src_v, dst_v, bufs, zbuf, ones_v, av, dv,
             rv, acc, gsem, ssem, dacc) = refs
        else:
            (out_hbm, src_v, dst_v, bufs, zbuf, ones_v, av, dv,
             rv, acc, gsem, ssem) = refs
            dacc = rdeg_out = None
        cid = lax.axis_index("c")
        sid = lax.axis_index("s")
        r0 = sid * _RPT
        t0 = _RPT * _NS
        data_slab = cid if with_deg else 0
        iota16 = lax.iota(jnp.int32, 16)
        zeros16 = jnp.zeros((16,), jnp.int32)

        @pl.loop(0, _ZR)
        def _fill(i):
            for c in range(_W // 16):
                zbuf[i, pl.ds(c * 16, 16)] = jnp.zeros((16,), jnp.float32)

        pltpu.sync_copy(src_hbm.at[2 * sid + cid], src_v)
        if with_deg:
            pltpu.sync_copy(dst_hbm.at[pl.ds(2 * sid, 2)], dst_v)
            pltpu.sync_copy(ones_hbm, ones_v)
            pltpu.sync_copy(zdeg_hbm.at[pl.ds(r0, _RPT)],
                            dacc.at[pl.ds(r0, _RPT)])

            @pl.when(sid == 0)
            def _zdtail():
                pltpu.sync_copy(zdeg_hbm.at[pl.ds(t0, _TAIL)],
                                dacc.at[pl.ds(t0, _TAIL)])
        else:
            pltpu.sync_copy(dst_hbm.at[pl.ds(2 * sid + cid, 1)], dst_v)
            pltpu.sync_copy(rdeg_hbm.at[cid].at[pl.ds(_RROW * sid, _GPT)],
                            rv.at[pl.ds(0, _GPT)])

        def issue_gather(y_hbm, k, j):
            pltpu.async_copy(y_hbm.at[src_v.at[k]], bufs[j], gsem[j])

        def issue_scatter(k, j, deg_too):
            pltpu.async_copy(bufs[j], acc.at[dst_v.at[data_slab].at[k]],
                             ssem[j], add=True)
            if deg_too:
                for w in range(2):
                    pltpu.async_copy(ones_v, dacc.at[dst_v.at[w].at[k]],
                                     ssem[j], add=True)

        def wait_gather(y_hbm, j):
            pltpu.make_async_copy(y_hbm.at[pl.ds(0, _C)], bufs[j],
                                  gsem[j]).wait()

        def wait_scatter(y_hbm, j, deg_too):
            pltpu.make_async_copy(y_hbm.at[pl.ds(0, _C)], bufs[j],
                                  ssem[j]).wait()
            if deg_too:
                for _ in range(2):
                    pltpu.make_async_copy(ones_hbm, ones_v, ssem[j]).wait()

        def scale_rows(row0, rvoff, ngroups, compute_recip):
            pltpu.sync_copy(acc.at[pl.ds(row0, ngroups * 16)],
                            av.at[pl.ds(0, ngroups * 16)])
            if compute_recip:
                pltpu.sync_copy(dacc.at[pl.ds(row0, ngroups * 16)],
                                dv.at[pl.ds(0, ngroups * 16)])

            @pl.loop(0, ngroups)
            def grp(g):
                if compute_recip:
                    dvals = plsc.load_gather(dv, [g * 16 + iota16, zeros16])
                    rv[rvoff + g] = 1.0 / jnp.maximum(dvals, 1.0)

                @pl.loop(0, 16)
                def node(t):
                    s = plsc.load_gather(
                        rv, [jnp.full((16,), rvoff + g, jnp.int32),
                             jnp.full((16,), t, jnp.int32)])
                    row = g * 16 + t
                    for c in range(_W // 16):
                        sl = pl.ds(c * 16, 16)
                        av[row, sl] = av[row, sl] * s

        for p in range(nph):
            y_hbm = y_hbms[p]
            deg_too = with_deg and p == 0
            first_recip = with_deg and p == 0

            for t in range(_RPT // _ZR):
                pltpu.sync_copy(zbuf, acc.at[pl.ds(r0 + t * _ZR, _ZR)])

            @pl.when(sid == 0)
            def _zero_tail():
                pltpu.sync_copy(zbuf.at[pl.ds(0, _TAIL)],
                                acc.at[pl.ds(t0, _TAIL)])

            plsc.subcore_barrier()

            for j in range(_NSLOT):
                issue_gather(y_hbm, j, j)

            @pl.loop(0, (_NCH - _NSLOT) // _NSLOT)
            def group(h):
                base = h * _NSLOT
                for bank in (0, 1):
                    for t in (0, 1):
                        j = 2 * bank + t
                        wait_gather(y_hbm, j)
                        issue_scatter(base + j, j, deg_too)
                    for t in (0, 1):
                        j = 2 * bank + t
                        wait_scatter(y_hbm, j, deg_too)
                        issue_gather(y_hbm, base + _NSLOT + j, j)

            for j in range(_NSLOT):
                wait_gather(y_hbm, j)
                issue_scatter(_NCH - _NSLOT + j, j, deg_too)
            for j in range(_NSLOT):
                wait_scatter(y_hbm, j, deg_too)

            plsc.subcore_barrier()

            scale_rows(r0, 0, _GPT, first_recip)
            pltpu.sync_copy(av, out_hbm.at[p].at[cid].at[pl.ds(r0, _RPT)])
            if first_recip:
                pltpu.sync_copy(
                    rv.at[pl.ds(0, _GPT)],
                    rdeg_out.at[cid].at[pl.ds(_RROW * sid, _GPT)])

            @pl.when(sid == 0)
            def _tail():
                if first_recip:
                    scale_rows(t0, _GPT, _TAIL // 16, True)
                    pltpu.sync_copy(
                        rv.at[pl.ds(_GPT, _TAIL // 16)],
                        rdeg_out.at[cid].at[pl.ds(_NS * _RROW,
                                                  _TAIL // 16)])
                else:
                    if not with_deg and p == 0:
                        pltpu.sync_copy(
                            rdeg_hbm.at[cid].at[pl.ds(_NS * _RROW, 1)],
                            rv.at[pl.ds(_GPT, 1)])
                    scale_rows(t0, _GPT, _TAIL // 16, False)
                pltpu.sync_copy(av.at[pl.ds(0, _TAIL)],
                                out_hbm.at[p].at[cid].at[pl.ds(t0, _TAIL)])

    zdeg = jnp.zeros((_N, _DW), jnp.float32)
    ones = jnp.ones((_C, _DW), jnp.float32)
    if rdeg_in is None:
        rdeg_in = jnp.zeros((_NC, _RD, 16), jnp.float32)
    k = pl.kernel(body, out_type=out_type, mesh=mesh, scratch_types=scratch,
                  compiler_params=pltpu.CompilerParams(
                      use_tc_tiling_on_sc=False,
                      needs_layout_passes=False))
    return k(*ys, src3, dst3, zdeg, ones, rdeg_in)


_Q = _N // 4


def _tc1_body(x4_ref, wa_ref, wb_ref, wra_ref, wrb_ref,
              y1a_ref, y1b_ref, r1a_ref, r1b_ref):
    xb = x4_ref[...]
    y1a_ref[...] = jnp.dot(xb, wa_ref[...], preferred_element_type=jnp.float32)
    y1b_ref[...] = jnp.dot(xb, wb_ref[...], preferred_element_type=jnp.float32)
    r1a_ref[...] = jnp.dot(xb, wra_ref[...],
                           preferred_element_type=jnp.float32)
    r1b_ref[...] = jnp.dot(xb, wrb_ref[...],
                           preferred_element_type=jnp.float32)


def _tc2_body(pm_ref, r1a_ref, r1b_ref, b1a_ref, b1b_ref,
              wla_ref, wlb_ref, wra_ref, wrb_ref, y2_ref, r2_ref):
    v = pm_ref[...].reshape(2, 2, _Q, 128)
    ha = jnp.maximum(v[0, 0] + v[0, 1] + r1a_ref[...] + b1a_ref[...], 0.0)
    hb = jnp.maximum(v[1, 0] + v[1, 1] + r1b_ref[...] + b1b_ref[...], 0.0)
    y2_ref[...] = (
        jnp.dot(ha, wla_ref[...], preferred_element_type=jnp.float32)
        + jnp.dot(hb, wlb_ref[...], preferred_element_type=jnp.float32))
    r2_ref[...] = (
        jnp.dot(ha, wra_ref[...], preferred_element_type=jnp.float32)
        + jnp.dot(hb, wrb_ref[...], preferred_element_type=jnp.float32))


def _tc3_body(pm_ref, r2_ref, b2_ref, z_ref):
    v = pm_ref[...].reshape(2, _Q, 128)
    z_ref[...] = jnp.maximum(v[0] + v[1] + r2_ref[...] + b2_ref[...], 0.0)


def kernel(x, edge_index, W_l1, W_r1, b1, W_l2, W_r2, b2):
    src3 = edge_index[0].reshape(_NW, _NCH, _C)
    dst3 = edge_index[1].reshape(_NW, _NCH, _C)

    eye4 = jnp.eye(4, dtype=jnp.float32)
    x4 = x.reshape(_Q, 4 * _D)
    wa = jnp.kron(eye4, W_l1[:, :_W])
    wb = jnp.kron(eye4, W_l1[:, _W:])
    wra = jnp.kron(eye4, W_r1[:, :_W])
    wrb = jnp.kron(eye4, W_r1[:, _W:])

    def full2d(shape):
        return pl.BlockSpec(shape, lambda i: (0, 0))

    q128 = jax.ShapeDtypeStruct((_Q, 128), jnp.float32)
    y1a, y1b, r1a, r1b = pl.pallas_call(
        _tc1_body,
        grid=(1,),
        in_specs=[full2d((_Q, 4 * _D))] + [full2d((4 * _D, 128))] * 4,
        out_specs=[full2d((_Q, 128))] * 4,
        out_shape=[q128] * 4,
    )(x4, wa, wb, wra, wrb)

    pm1, rdeg = _sc_agg([y1a.reshape(_N, _W), y1b.reshape(_N, _W)],
                        src3, dst3, None)

    b1a = jnp.tile(b1[:_W], 4).reshape(1, 128)
    b1b = jnp.tile(b1[_W:], 4).reshape(1, 128)
    wla = jnp.kron(eye4, W_l2[:_W, :])
    wlb = jnp.kron(eye4, W_l2[_W:, :])
    wr2a = jnp.kron(eye4, W_r2[:_W, :])
    wr2b = jnp.kron(eye4, W_r2[_W:, :])
    y2, r2 = pl.pallas_call(
        _tc2_body,
        grid=(1,),
        in_specs=[full2d((4 * _Q, 128)), full2d((_Q, 128)),
                  full2d((_Q, 128)), full2d((1, 128)), full2d((1, 128))]
                 + [full2d((128, 128))] * 4,
        out_specs=[full2d((_Q, 128))] * 2,
        out_shape=[q128] * 2,
    )(pm1.reshape(4 * _Q, 128), r1a, r1b, b1a, b1b,
      wla, wlb, wr2a, wr2b)

    pm2, = _sc_agg([y2.reshape(_N, _Z)], src3, dst3, rdeg)

    b2t = jnp.tile(b2, 4).reshape(1, 128)
    zp = pl.pallas_call(
        _tc3_body,
        grid=(1,),
        in_specs=[full2d((2 * _Q, 128)), full2d((_Q, 128)),
                  full2d((1, 128))],
        out_specs=full2d((_Q, 128)),
        out_shape=q128,
    )(pm2.reshape(2 * _Q, 128), r2, b2t)

    return zp.reshape(_N, _Z)


# speedup vs baseline: 1.1745x; 1.0062x over previous
import jax
import jax.numpy as jnp
from jax import lax
from jax.experimental import pallas as pl
from jax.experimental.pallas import tpu as pltpu
from jax.experimental.pallas import tpu_sc as plsc

_N = 10000
_E = 320000
_D = 128
_H = 64
_Z = 32

_NC = 2
_NS = 16
_NW = _NC * _NS
_EPW = _E // _NW
_C = 125
_NCH = _EPW // _C
_NSLOT = 4
_RPT = 624
_TAIL = _N - _RPT * _NS
_W = 32

_DW = 8
_GPT = _RPT // 16
_RROW = 40
_RD = _NS * _RROW + 8
_ZR = 104


def _sc_agg(ys, src3, dst3, rdeg_in):
    mesh = plsc.VectorSubcoreMesh(core_axis_name="c", subcore_axis_name="s")
    with_deg = rdeg_in is None
    nsl = len(ys)

    out_type = [jax.ShapeDtypeStruct((_NC, _N, _W), jnp.float32)] * nsl
    if with_deg:
        out_type.append(jax.ShapeDtypeStruct((_NC, _RD, 16), jnp.float32))

    scratch = [
        pltpu.VMEM((_NCH, _C), jnp.int32),
        pltpu.VMEM((2 if with_deg else 1, _NCH, _C), jnp.int32),
        [pltpu.VMEM((_C, _W), jnp.float32)] * _NSLOT,
        pltpu.VMEM((_ZR, _W), jnp.float32),
        pltpu.VMEM((_C, _DW), jnp.float32),
        pltpu.VMEM((_RPT, _W), jnp.float32),
        pltpu.VMEM((_RPT, _DW), jnp.float32),
        pltpu.VMEM((_RROW, 16), jnp.float32),
        [pltpu.VMEM_SHARED((_N, _W), jnp.float32)] * nsl,
        [pltpu.SemaphoreType.DMA] * _NSLOT,
        [pltpu.SemaphoreType.DMA] * _NSLOT,
    ]
    if with_deg:
        scratch.append(
            pltpu.VMEM_SHARED((_N, _DW), jnp.float32))

    def body(*args):
        y_hbms = args[:nsl]
        (src_hbm, dst_hbm, zdeg_hbm, ones_hbm, rdeg_hbm) = args[nsl:nsl + 5]
        refs = args[nsl + 5:]
        out_hbms = refs[:nsl]
        if with_deg:
            (rdeg_out, src_v, dst_v, bufs, zbuf, ones_v, av, dv,
             rv, accs, gsem, ssem, dacc) = refs[nsl:]
        else:
            (src_v, dst_v, bufs, zbuf, ones_v, av, dv,
             rv, accs, gsem, ssem) = refs[nsl:]
            dacc = rdeg_out = None
        cid = lax.axis_index("c")
        sid = lax.axis_index("s")
        r0 = sid * _RPT
        t0 = _RPT * _NS
        data_slab = cid if with_deg else 0
        iota16 = lax.iota(jnp.int32, 16)
        zeros16 = jnp.zeros((16,), jnp.int32)

        @pl.loop(0, _ZR)
        def _fill(i):
            for c in range(_W // 16):
                zbuf[i, pl.ds(c * 16, 16)] = jnp.zeros((16,), jnp.float32)

        for acc in accs:
            for t in range(_RPT // _ZR):
                pltpu.sync_copy(zbuf, acc.at[pl.ds(r0 + t * _ZR, _ZR)])
        pltpu.sync_copy(src_hbm.at[2 * sid + cid], src_v)
        if with_deg:
            pltpu.sync_copy(dst_hbm.at[pl.ds(2 * sid, 2)], dst_v)
            pltpu.sync_copy(ones_hbm, ones_v)
            pltpu.sync_copy(zdeg_hbm.at[pl.ds(r0, _RPT)],
                            dacc.at[pl.ds(r0, _RPT)])
        else:
            pltpu.sync_copy(dst_hbm.at[pl.ds(2 * sid + cid, 1)], dst_v)
            pltpu.sync_copy(rdeg_hbm.at[cid].at[pl.ds(_RROW * sid, _GPT)],
                            rv.at[pl.ds(0, _GPT)])

        @pl.when(sid == 0)
        def _zero_tail():
            for acc in accs:
                pltpu.sync_copy(zbuf.at[pl.ds(0, _TAIL)],
                                acc.at[pl.ds(t0, _TAIL)])
            if with_deg:
                pltpu.sync_copy(zdeg_hbm.at[pl.ds(t0, _TAIL)],
                                dacc.at[pl.ds(t0, _TAIL)])

        plsc.subcore_barrier()

        _VC = _NCH * nsl

        def issue_gather(vc, j):
            pltpu.async_copy(y_hbms[j % nsl].at[src_v.at[vc // nsl]],
                             bufs[j], gsem[j])

        def issue_scatter(vc, j):
            k = vc // nsl
            pltpu.async_copy(bufs[j],
                             accs[j % nsl].at[dst_v.at[data_slab].at[k]],
                             ssem[j], add=True)
            if with_deg and j % nsl == 0:
                for w in range(2):
                    pltpu.async_copy(ones_v, dacc.at[dst_v.at[w].at[k]],
                                     ssem[j], add=True)

        def wait_gather(j):
            pltpu.make_async_copy(y_hbms[0].at[pl.ds(0, _C)], bufs[j],
                                  gsem[j]).wait()

        def wait_scatter(j):
            pltpu.make_async_copy(y_hbms[0].at[pl.ds(0, _C)], bufs[j],
                                  ssem[j]).wait()
            if with_deg and j % nsl == 0:
                for _ in range(2):
                    pltpu.make_async_copy(ones_hbm, ones_v, ssem[j]).wait()

        for j in range(_NSLOT):
            issue_gather(j, j)

        @pl.loop(0, (_VC - _NSLOT) // _NSLOT)
        def group(h):
            base = h * _NSLOT
            for bank in (0, 1):
                for t in (0, 1):
                    j = 2 * bank + t
                    wait_gather(j)
                    issue_scatter(base + j, j)
                for t in (0, 1):
                    j = 2 * bank + t
                    wait_scatter(j)
                    issue_gather(base + _NSLOT + j, j)

        for j in range(_NSLOT):
            wait_gather(j)
            issue_scatter(_VC - _NSLOT + j, j)
        for j in range(_NSLOT):
            wait_scatter(j)

        plsc.subcore_barrier()

        def scale_rows(acc, row0, rvoff, ngroups, compute_recip):
            pltpu.sync_copy(acc.at[pl.ds(row0, ngroups * 16)],
                            av.at[pl.ds(0, ngroups * 16)])
            if compute_recip:
                pltpu.sync_copy(dacc.at[pl.ds(row0, ngroups * 16)],
                                dv.at[pl.ds(0, ngroups * 16)])

            @pl.loop(0, ngroups)
            def grp(g):
                if compute_recip:
                    dvals = plsc.load_gather(dv, [g * 16 + iota16, zeros16])
                    rv[rvoff + g] = 1.0 / jnp.maximum(dvals, 1.0)

                @pl.loop(0, 16)
                def node(t):
                    s = plsc.load_gather(
                        rv, [jnp.full((16,), rvoff + g, jnp.int32),
                             jnp.full((16,), t, jnp.int32)])
                    row = g * 16 + t
                    for c in range(_W // 16):
                        sl = pl.ds(c * 16, 16)
                        av[row, sl] = av[row, sl] * s

        for p in range(nsl):
            first = p == 0
            scale_rows(accs[p], r0, 0, _GPT, with_deg and first)
            pltpu.sync_copy(av, out_hbms[p].at[cid].at[pl.ds(r0, _RPT)])
            if with_deg and first:
                pltpu.sync_copy(
                    rv.at[pl.ds(0, _GPT)],
                    rdeg_out.at[cid].at[pl.ds(_RROW * sid, _GPT)])

            @pl.when(sid == 0)
            def _tail():
                if with_deg and first:
                    scale_rows(accs[p], t0, _GPT, _TAIL // 16, True)
                    pltpu.sync_copy(
                        rv.at[pl.ds(_GPT, _TAIL // 16)],
                        rdeg_out.at[cid].at[pl.ds(_NS * _RROW,
                                                  _TAIL // 16)])
                else:
                    if not with_deg and first:
                        pltpu.sync_copy(
                            rdeg_hbm.at[cid].at[pl.ds(_NS * _RROW, 1)],
                            rv.at[pl.ds(_GPT, 1)])
                    scale_rows(accs[p], t0, _GPT, _TAIL // 16, False)
                pltpu.sync_copy(
                    av.at[pl.ds(0, _TAIL)],
                    out_hbms[p].at[cid].at[pl.ds(t0, _TAIL)])

    zdeg = jnp.zeros((_N, _DW), jnp.float32)
    ones = jnp.ones((_C, _DW), jnp.float32)
    if rdeg_in is None:
        rdeg_in = jnp.zeros((_NC, _RD, 16), jnp.float32)
    k = pl.kernel(body, out_type=out_type, mesh=mesh, scratch_types=scratch,
                  compiler_params=pltpu.CompilerParams(
                      use_tc_tiling_on_sc=False,
                      needs_layout_passes=False))
    return k(*ys, src3, dst3, zdeg, ones, rdeg_in)


_Q = _N // 4


def _tc1_body(x4_ref, wa_ref, wb_ref, wra_ref, wrb_ref,
              y1a_ref, y1b_ref, r1a_ref, r1b_ref):
    xb = x4_ref[...]
    y1a_ref[...] = jnp.dot(xb, wa_ref[...], preferred_element_type=jnp.float32)
    y1b_ref[...] = jnp.dot(xb, wb_ref[...], preferred_element_type=jnp.float32)
    r1a_ref[...] = jnp.dot(xb, wra_ref[...],
                           preferred_element_type=jnp.float32)
    r1b_ref[...] = jnp.dot(xb, wrb_ref[...],
                           preferred_element_type=jnp.float32)


def _tc2_body(pma_ref, pmb_ref, r1a_ref, r1b_ref, b1a_ref, b1b_ref,
              wla_ref, wlb_ref, wra_ref, wrb_ref, y2_ref, r2_ref):
    va = pma_ref[...].reshape(2, _Q, 128)
    vb = pmb_ref[...].reshape(2, _Q, 128)
    ha = jnp.maximum(va[0] + va[1] + r1a_ref[...] + b1a_ref[...], 0.0)
    hb = jnp.maximum(vb[0] + vb[1] + r1b_ref[...] + b1b_ref[...], 0.0)
    y2_ref[...] = (
        jnp.dot(ha, wla_ref[...], preferred_element_type=jnp.float32)
        + jnp.dot(hb, wlb_ref[...], preferred_element_type=jnp.float32))
    r2_ref[...] = (
        jnp.dot(ha, wra_ref[...], preferred_element_type=jnp.float32)
        + jnp.dot(hb, wrb_ref[...], preferred_element_type=jnp.float32))


def _tc3_body(pm_ref, r2_ref, b2_ref, z_ref):
    v = pm_ref[...].reshape(2, _Q, 128)
    z_ref[...] = jnp.maximum(v[0] + v[1] + r2_ref[...] + b2_ref[...], 0.0)


def kernel(x, edge_index, W_l1, W_r1, b1, W_l2, W_r2, b2):
    src3 = edge_index[0].reshape(_NW, _NCH, _C)
    dst3 = edge_index[1].reshape(_NW, _NCH, _C)

    eye4 = jnp.eye(4, dtype=jnp.float32)
    x4 = x.reshape(_Q, 4 * _D)
    wa = jnp.kron(eye4, W_l1[:, :_W])
    wb = jnp.kron(eye4, W_l1[:, _W:])
    wra = jnp.kron(eye4, W_r1[:, :_W])
    wrb = jnp.kron(eye4, W_r1[:, _W:])

    def full2d(shape):
        return pl.BlockSpec(shape, lambda i: (0, 0))

    q128 = jax.ShapeDtypeStruct((_Q, 128), jnp.float32)
    y1a, y1b, r1a, r1b = pl.pallas_call(
        _tc1_body,
        grid=(1,),
        in_specs=[full2d((_Q, 4 * _D))] + [full2d((4 * _D, 128))] * 4,
        out_specs=[full2d((_Q, 128))] * 4,
        out_shape=[q128] * 4,
    )(x4, wa, wb, wra, wrb)

    pm1a, pm1b, rdeg = _sc_agg([y1a.reshape(_N, _W), y1b.reshape(_N, _W)],
                               src3, dst3, None)

    b1a = jnp.tile(b1[:_W], 4).reshape(1, 128)
    b1b = jnp.tile(b1[_W:], 4).reshape(1, 128)
    wla = jnp.kron(eye4, W_l2[:_W, :])
    wlb = jnp.kron(eye4, W_l2[_W:, :])
    wr2a = jnp.kron(eye4, W_r2[:_W, :])
    wr2b = jnp.kron(eye4, W_r2[_W:, :])
    y2, r2 = pl.pallas_call(
        _tc2_body,
        grid=(1,),
        in_specs=[full2d((2 * _Q, 128)), full2d((2 * _Q, 128)),
                  full2d((_Q, 128)), full2d((_Q, 128)),
                  full2d((1, 128)), full2d((1, 128))]
                 + [full2d((128, 128))] * 4,
        out_specs=[full2d((_Q, 128))] * 2,
        out_shape=[q128] * 2,
    )(pm1a.reshape(2 * _Q, 128), pm1b.reshape(2 * _Q, 128), r1a, r1b,
      b1a, b1b, wla, wlb, wr2a, wr2b)

    pm2, = _sc_agg([y2.reshape(_N, _Z)], src3, dst3, rdeg)

    b2t = jnp.tile(b2, 4).reshape(1, 128)
    zp = pl.pallas_call(
        _tc3_body,
        grid=(1,),
        in_specs=[full2d((2 * _Q, 128)), full2d((_Q, 128)),
                  full2d((1, 128))],
        out_specs=full2d((_Q, 128)),
        out_shape=q128,
    )(pm2.reshape(2 * _Q, 128), r2, b2t)

    return zp.reshape(_N, _Z)


# speedup vs baseline: 1.1750x; 1.0004x over previous
import jax
import jax.numpy as jnp
from jax import lax
from jax.experimental import pallas as pl
from jax.experimental.pallas import tpu as pltpu
from jax.experimental.pallas import tpu_sc as plsc

_N = 10000
_E = 320000
_D = 128
_H = 64
_Z = 32

_NC = 2
_NS = 16
_NW = _NC * _NS
_EPW = _E // _NW
_C = 125
_NCH = _EPW // _C
_NSLOT = 4
_RPT = 624
_TAIL = _N - _RPT * _NS
_W = 32

_DW = 8
_GPT = _RPT // 16
_RROW = 40
_RD = _NS * _RROW + 8
_ZR = 104


def _sc_agg(ys, src3, dst3, rdeg_in):
    mesh = plsc.VectorSubcoreMesh(core_axis_name="c", subcore_axis_name="s")
    with_deg = rdeg_in is None
    nsl = len(ys)

    out_type = [jax.ShapeDtypeStruct((_NC, _N, _W), jnp.float32)] * nsl
    if with_deg:
        out_type.append(jax.ShapeDtypeStruct((_NC, _RD, 16), jnp.float32))

    scratch = [
        pltpu.VMEM((_NCH, _C), jnp.int32),
        pltpu.VMEM((2 if with_deg else 1, _NCH, _C), jnp.int32),
        [pltpu.VMEM((_C, _W), jnp.float32)] * _NSLOT,
        pltpu.VMEM((_ZR, _W), jnp.float32),
        pltpu.VMEM((_C, _DW), jnp.float32),
        pltpu.VMEM((_RPT, _W), jnp.float32),
        pltpu.VMEM((_RPT, _DW), jnp.float32),
        pltpu.VMEM((_RROW, 16), jnp.float32),
        [pltpu.VMEM_SHARED((_N, _W), jnp.float32)] * nsl,
        [pltpu.SemaphoreType.DMA] * _NSLOT,
        [pltpu.SemaphoreType.DMA] * _NSLOT,
    ]
    if with_deg:
        scratch.append(
            pltpu.VMEM_SHARED((_N, _DW), jnp.float32))

    def body(*args):
        y_hbms = args[:nsl]
        (src_hbm, dst_hbm, zdeg_hbm, ones_hbm, rdeg_hbm) = args[nsl:nsl + 5]
        refs = args[nsl + 5:]
        out_hbms = refs[:nsl]
        if with_deg:
            (rdeg_out, src_v, dst_v, bufs, zbuf, ones_v, av, dv,
             rv, accs, gsem, ssem, dacc) = refs[nsl:]
        else:
            (src_v, dst_v, bufs, zbuf, ones_v, av, dv,
             rv, accs, gsem, ssem) = refs[nsl:]
            dacc = rdeg_out = None
        cid = lax.axis_index("c")
        sid = lax.axis_index("s")
        r0 = sid * _RPT
        t0 = _RPT * _NS
        data_slab = cid if with_deg else 0
        iota16 = lax.iota(jnp.int32, 16)
        zeros16 = jnp.zeros((16,), jnp.int32)

        @pl.loop(0, _ZR)
        def _fill(i):
            for c in range(_W // 16):
                zbuf[i, pl.ds(c * 16, 16)] = jnp.zeros((16,), jnp.float32)

        for acc in accs:
            for t in range(_RPT // _ZR):
                pltpu.sync_copy(zbuf, acc.at[pl.ds(r0 + t * _ZR, _ZR)])
        pltpu.sync_copy(src_hbm.at[2 * sid + cid], src_v)
        if with_deg:
            pltpu.sync_copy(dst_hbm.at[pl.ds(2 * sid, 2)], dst_v)
            pltpu.sync_copy(ones_hbm, ones_v)
            pltpu.sync_copy(zdeg_hbm.at[pl.ds(r0, _RPT)],
                            dacc.at[pl.ds(r0, _RPT)])
        else:
            pltpu.sync_copy(dst_hbm.at[pl.ds(2 * sid + cid, 1)], dst_v)
            pltpu.sync_copy(rdeg_hbm.at[cid].at[pl.ds(_RROW * sid, _GPT)],
                            rv.at[pl.ds(0, _GPT)])

        @pl.when(sid == 0)
        def _zero_tail():
            for acc in accs:
                pltpu.sync_copy(zbuf.at[pl.ds(0, _TAIL)],
                                acc.at[pl.ds(t0, _TAIL)])
            if with_deg:
                pltpu.sync_copy(zdeg_hbm.at[pl.ds(t0, _TAIL)],
                                dacc.at[pl.ds(t0, _TAIL)])

        plsc.subcore_barrier()

        _VC = _NCH * nsl

        def issue_gather(vc, j):
            pltpu.async_copy(y_hbms[j % nsl].at[src_v.at[vc // nsl]],
                             bufs[j], gsem[j])

        def issue_scatter(vc, j):
            k = vc // nsl
            pltpu.async_copy(bufs[j],
                             accs[j % nsl].at[dst_v.at[data_slab].at[k]],
                             ssem[j], add=True)
            if with_deg and j % nsl == 0:
                for w in range(2):
                    pltpu.async_copy(ones_v, dacc.at[dst_v.at[w].at[k]],
                                     ssem[j], add=True)

        def wait_gather(j):
            pltpu.make_async_copy(y_hbms[0].at[pl.ds(0, _C)], bufs[j],
                                  gsem[j]).wait()

        def wait_scatter(j):
            pltpu.make_async_copy(y_hbms[0].at[pl.ds(0, _C)], bufs[j],
                                  ssem[j]).wait()
            if with_deg and j % nsl == 0:
                for _ in range(2):
                    pltpu.make_async_copy(ones_hbm, ones_v, ssem[j]).wait()

        for j in range(_NSLOT):
            issue_gather(j, j)

        @pl.loop(0, (_VC - _NSLOT) // _NSLOT)
        def group(h):
            base = h * _NSLOT
            for bank in (0, 1):
                for t in range(_NSLOT // 2):
                    j = (_NSLOT // 2) * bank + t
                    wait_gather(j)
                    issue_scatter(base + j, j)
                for t in range(_NSLOT // 2):
                    j = (_NSLOT // 2) * bank + t
                    wait_scatter(j)
                    issue_gather(base + _NSLOT + j, j)

        for j in range(_NSLOT):
            wait_gather(j)
            issue_scatter(_VC - _NSLOT + j, j)
        for j in range(_NSLOT):
            wait_scatter(j)

        plsc.subcore_barrier()

        def scale_rows(acc, row0, rvoff, ngroups, compute_recip):
            pltpu.sync_copy(acc.at[pl.ds(row0, ngroups * 16)],
                            av.at[pl.ds(0, ngroups * 16)])
            if compute_recip:
                pltpu.sync_copy(dacc.at[pl.ds(row0, ngroups * 16)],
                                dv.at[pl.ds(0, ngroups * 16)])

            @pl.loop(0, ngroups)
            def grp(g):
                if compute_recip:
                    dvals = plsc.load_gather(dv, [g * 16 + iota16, zeros16])
                    rv[rvoff + g] = 1.0 / jnp.maximum(dvals, 1.0)

                @pl.loop(0, 16)
                def node(t):
                    s = plsc.load_gather(
                        rv, [jnp.full((16,), rvoff + g, jnp.int32),
                             jnp.full((16,), t, jnp.int32)])
                    row = g * 16 + t
                    for c in range(_W // 16):
                        sl = pl.ds(c * 16, 16)
                        av[row, sl] = av[row, sl] * s

        for p in range(nsl):
            first = p == 0
            scale_rows(accs[p], r0, 0, _GPT, with_deg and first)
            pltpu.sync_copy(av, out_hbms[p].at[cid].at[pl.ds(r0, _RPT)])
            if with_deg and first:
                pltpu.sync_copy(
                    rv.at[pl.ds(0, _GPT)],
                    rdeg_out.at[cid].at[pl.ds(_RROW * sid, _GPT)])

            @pl.when(sid == 0)
            def _tail():
                if with_deg and first:
                    scale_rows(accs[p], t0, _GPT, _TAIL // 16, True)
                    pltpu.sync_copy(
                        rv.at[pl.ds(_GPT, _TAIL // 16)],
                        rdeg_out.at[cid].at[pl.ds(_NS * _RROW,
                                                  _TAIL // 16)])
                else:
                    if not with_deg and first:
                        pltpu.sync_copy(
                            rdeg_hbm.at[cid].at[pl.ds(_NS * _RROW, 1)],
                            rv.at[pl.ds(_GPT, 1)])
                    scale_rows(accs[p], t0, _GPT, _TAIL // 16, False)
                pltpu.sync_copy(
                    av.at[pl.ds(0, _TAIL)],
                    out_hbms[p].at[cid].at[pl.ds(t0, _TAIL)])

    zdeg = jnp.zeros((_N, _DW), jnp.float32)
    ones = jnp.ones((_C, _DW), jnp.float32)
    if rdeg_in is None:
        rdeg_in = jnp.zeros((_NC, _RD, 16), jnp.float32)
    k = pl.kernel(body, out_type=out_type, mesh=mesh, scratch_types=scratch,
                  compiler_params=pltpu.CompilerParams(
                      use_tc_tiling_on_sc=False,
                      needs_layout_passes=False))
    return k(*ys, src3, dst3, zdeg, ones, rdeg_in)


_Q = _N // 4


def _tc1_body(x4_ref, wa_ref, wb_ref, wra_ref, wrb_ref,
              y1a_ref, y1b_ref, r1a_ref, r1b_ref):
    xb = x4_ref[...]
    y1a_ref[...] = jnp.dot(xb, wa_ref[...], preferred_element_type=jnp.float32)
    y1b_ref[...] = jnp.dot(xb, wb_ref[...], preferred_element_type=jnp.float32)
    r1a_ref[...] = jnp.dot(xb, wra_ref[...],
                           preferred_element_type=jnp.float32)
    r1b_ref[...] = jnp.dot(xb, wrb_ref[...],
                           preferred_element_type=jnp.float32)


def _tc2_body(pma_ref, pmb_ref, r1a_ref, r1b_ref, b1a_ref, b1b_ref,
              wla_ref, wlb_ref, wra_ref, wrb_ref, y2_ref, r2_ref):
    va = pma_ref[...].reshape(2, _Q, 128)
    vb = pmb_ref[...].reshape(2, _Q, 128)
    ha = jnp.maximum(va[0] + va[1] + r1a_ref[...] + b1a_ref[...], 0.0)
    hb = jnp.maximum(vb[0] + vb[1] + r1b_ref[...] + b1b_ref[...], 0.0)
    y2_ref[...] = (
        jnp.dot(ha, wla_ref[...], preferred_element_type=jnp.float32)
        + jnp.dot(hb, wlb_ref[...], preferred_element_type=jnp.float32))
    r2_ref[...] = (
        jnp.dot(ha, wra_ref[...], preferred_element_type=jnp.float32)
        + jnp.dot(hb, wrb_ref[...], preferred_element_type=jnp.float32))


def _tc3_body(pm_ref, r2_ref, b2_ref, z_ref):
    v = pm_ref[...].reshape(2, _Q, 128)
    z_ref[...] = jnp.maximum(v[0] + v[1] + r2_ref[...] + b2_ref[...], 0.0)


def kernel(x, edge_index, W_l1, W_r1, b1, W_l2, W_r2, b2):
    src3 = edge_index[0].reshape(_NW, _NCH, _C)
    dst3 = edge_index[1].reshape(_NW, _NCH, _C)

    eye4 = jnp.eye(4, dtype=jnp.float32)
    x4 = x.reshape(_Q, 4 * _D)
    wa = jnp.kron(eye4, W_l1[:, :_W])
    wb = jnp.kron(eye4, W_l1[:, _W:])
    wra = jnp.kron(eye4, W_r1[:, :_W])
    wrb = jnp.kron(eye4, W_r1[:, _W:])

    def full2d(shape):
        return pl.BlockSpec(shape, lambda i: (0, 0))

    q128 = jax.ShapeDtypeStruct((_Q, 128), jnp.float32)
    y1a, y1b, r1a, r1b = pl.pallas_call(
        _tc1_body,
        grid=(1,),
        in_specs=[full2d((_Q, 4 * _D))] + [full2d((4 * _D, 128))] * 4,
        out_specs=[full2d((_Q, 128))] * 4,
        out_shape=[q128] * 4,
    )(x4, wa, wb, wra, wrb)

    pm1a, pm1b, rdeg = _sc_agg([y1a.reshape(_N, _W), y1b.reshape(_N, _W)],
                               src3, dst3, None)

    b1a = jnp.tile(b1[:_W], 4).reshape(1, 128)
    b1b = jnp.tile(b1[_W:], 4).reshape(1, 128)
    wla = jnp.kron(eye4, W_l2[:_W, :])
    wlb = jnp.kron(eye4, W_l2[_W:, :])
    wr2a = jnp.kron(eye4, W_r2[:_W, :])
    wr2b = jnp.kron(eye4, W_r2[_W:, :])
    y2, r2 = pl.pallas_call(
        _tc2_body,
        grid=(1,),
        in_specs=[full2d((2 * _Q, 128)), full2d((2 * _Q, 128)),
                  full2d((_Q, 128)), full2d((_Q, 128)),
                  full2d((1, 128)), full2d((1, 128))]
                 + [full2d((128, 128))] * 4,
        out_specs=[full2d((_Q, 128))] * 2,
        out_shape=[q128] * 2,
    )(pm1a.reshape(2 * _Q, 128), pm1b.reshape(2 * _Q, 128), r1a, r1b,
      b1a, b1b, wla, wlb, wr2a, wr2b)

    pm2, = _sc_agg([y2.reshape(_N, _Z)], src3, dst3, rdeg)

    b2t = jnp.tile(b2, 4).reshape(1, 128)
    zp = pl.pallas_call(
        _tc3_body,
        grid=(1,),
        in_specs=[full2d((2 * _Q, 128)), full2d((_Q, 128)),
                  full2d((1, 128))],
        out_specs=full2d((_Q, 128)),
        out_shape=q128,
    )(pm2.reshape(2 * _Q, 128), r2, b2t)

    return zp.reshape(_N, _Z)
